# baseline (speedup 1.0000x reference)
import functools
import math

import jax
import jax.numpy as jnp
from jax import lax
from jax.experimental import pallas as pl
from jax.experimental.pallas import tpu as pltpu
from jax.experimental.pallas import tpu_sc as plsc

D = 64
LANES = 16
NC, NS = 2, 16
NW = NC * NS
RPG = 128
K = 5
CHUNK = K * RPG
SCALE = math.sqrt(D)


def _build(n_flat):
  assert n_flat % (NW * CHUNK) == 0
  per_w = n_flat // NW
  nch = per_w // CHUNK
  assert nch % 2 == 0
  mesh = plsc.VectorSubcoreMesh(core_axis_name="c", subcore_axis_name="s")

  @functools.partial(
      pl.kernel,
      out_type=jax.ShapeDtypeStruct((NW, per_w // RPG, RPG, D), jnp.float32),
      mesh=mesh,
      compiler_params=pltpu.CompilerParams(use_tc_tiling_on_sc=False),
      scratch_types=[
          pltpu.VMEM((1, per_w), jnp.int32),
          pltpu.VMEM((K, RPG, D), jnp.float32),
          pltpu.VMEM((K, RPG, D), jnp.float32),
          pltpu.SemaphoreType.DMA,
          pltpu.SemaphoreType.DMA,
      ],
  )
  def embed(x_hbm, table_hbm, out_hbm, idx_all, rows0, rows1, gsem0, gsem1):
    wid = lax.axis_index("s") * NC + lax.axis_index("c")
    rows_b = (rows0, rows1)
    gsem = (gsem0, gsem1)

    def fire_gather(g, b):
      for j in range(K):
        idx_sl = idx_all.at[0, pl.ds((g * K + j) * RPG, RPG)]
        pltpu.async_copy(table_hbm.at[idx_sl], rows_b[b].at[j], gsem[b])

    def drain_gather(b):
      for j in range(K):
        idx_sl = idx_all.at[0, pl.ds(j * RPG, RPG)]
        pltpu.make_async_copy(
            table_hbm.at[idx_sl], rows_b[b].at[j], gsem[b]).wait()

    def scale_store(g, b):
      r = rows_b[b]

      @pl.loop(0, RPG)
      def _(i):
        for j in range(K):
          for l in range(D // LANES):
            sl = pl.ds(l * LANES, LANES)
            r[j, i, sl] = r[j, i, sl] * SCALE

      pltpu.sync_copy(r, out_hbm.at[wid, pl.ds(g * K, K)])

    pltpu.sync_copy(x_hbm.at[wid], idx_all)
    fire_gather(0, 0)

    @pl.loop(0, nch // 2)
    def _(i):
      for b in range(2):
        g = 2 * i + b
        nb = 1 - b

        @pl.when(g + 1 < nch)
        def _():
          fire_gather(g + 1, nb)

        drain_gather(b)
        scale_store(g, b)

  return embed


@jax.jit
def kernel(x, table):
  B, L = x.shape
  n = B * L
  xr = x.astype(jnp.int32).reshape(NW, 1, n // NW)
  out = _build(n)(xr, table)
  return out.reshape(B, L, D)
